# Initial kernel scaffold; baseline (speedup 1.0000x reference)
import jax
import jax.numpy as jnp
from jax.experimental import pallas as pl


def kernel(x, Wq, Wk):
    raise NotImplementedError("write your pallas kernel here")


# speedup vs baseline: 7.6299x; 7.6299x over previous
import functools

import jax
import jax.numpy as jnp
from jax.experimental import pallas as pl
from jax.experimental.pallas import tpu as pltpu

_LOCAL_WINDOW = 32
_GLOBAL_K = 64
_ROW_BLOCK = 256


def _proj_body(x_ref, wq_ref, wk_ref, qp_ref, kp_ref):
    x = x_ref[0]
    dn = (((1,), (1,)), ((), ()))
    qp_ref[0] = jax.lax.dot_general(
        x, wq_ref[...], dn, precision=jax.lax.Precision.HIGHEST,
        preferred_element_type=jnp.float32)
    kp_ref[0] = jax.lax.dot_general(
        x, wk_ref[...], dn, precision=jax.lax.Precision.HIGHEST,
        preferred_element_type=jnp.float32)


def _mask_body(qp_ref, kp_ref, out_ref, *, seq_len, row_block, k):
    rb = pl.program_id(1)
    q = qp_ref[0]
    kp = kp_ref[0]
    dn = (((1,), (1,)), ((), ()))
    s = jax.lax.dot_general(q, kp, dn, precision=jax.lax.Precision.HIGHEST,
                            preferred_element_type=jnp.float32)
    s = jnp.maximum(s, 0.0)
    bits = jax.lax.bitcast_convert_type(s, jnp.int32) & 0x7FFFFFFF

    def count_ge(thresh):
        return jnp.sum((bits >= thresh).astype(jnp.int32), axis=1,
                       keepdims=True)

    lo = jnp.zeros((row_block, 1), jnp.int32)
    hi = jnp.full((row_block, 1), jnp.int32(0x7FFFFFFF))

    def val_step(_, carry):
        lo, hi = carry
        mid = lo + ((hi - lo) >> 1)
        ok = count_ge(mid) >= k
        return jnp.where(ok, mid, lo), jnp.where(ok, hi, mid)

    lo, hi = jax.lax.fori_loop(0, 31, val_step, (lo, hi))
    t = lo

    gt = bits > t
    n_gt = jnp.sum(gt.astype(jnp.int32), axis=1, keepdims=True)
    needed = k - n_gt
    tie = bits == t
    col = jax.lax.broadcasted_iota(jnp.int32, (row_block, seq_len), 1)

    lo2 = jnp.zeros((row_block, 1), jnp.int32)
    hi2 = jnp.full((row_block, 1), jnp.int32(seq_len))

    def idx_step(_, carry):
        lo2, hi2 = carry
        mid = lo2 + ((hi2 - lo2) >> 1)
        cnt = jnp.sum((tie & (col < mid)).astype(jnp.int32), axis=1,
                      keepdims=True)
        ok = cnt >= needed
        return jnp.where(ok, lo2, mid), jnp.where(ok, mid, hi2)

    lo2, hi2 = jax.lax.fori_loop(0, 12, idx_step, (lo2, hi2))
    selected = gt | (tie & (col < hi2))

    row = rb * row_block + jax.lax.broadcasted_iota(
        jnp.int32, (row_block, seq_len), 0)
    local = (col <= row) & (col >= row - (_LOCAL_WINDOW - 1))
    out_ref[0, 0] = jnp.where(local | selected, jnp.float32(0.0),
                              jnp.float32(-jnp.inf))


def kernel(x, Wq, Wk):
    B, S, D = x.shape
    idim = Wq.shape[0]
    qp, kp = pl.pallas_call(
        _proj_body,
        grid=(B,),
        in_specs=[
            pl.BlockSpec((1, S, D), lambda b: (b, 0, 0)),
            pl.BlockSpec((idim, D), lambda b: (0, 0)),
            pl.BlockSpec((idim, D), lambda b: (0, 0)),
        ],
        out_specs=[
            pl.BlockSpec((1, S, idim), lambda b: (b, 0, 0)),
            pl.BlockSpec((1, S, idim), lambda b: (b, 0, 0)),
        ],
        out_shape=[
            jax.ShapeDtypeStruct((B, S, idim), jnp.float32),
            jax.ShapeDtypeStruct((B, S, idim), jnp.float32),
        ],
        compiler_params=pltpu.CompilerParams(
            dimension_semantics=("arbitrary",)),
    )(x, Wq, Wk)

    R = _ROW_BLOCK
    body = functools.partial(_mask_body, seq_len=S, row_block=R,
                             k=min(_GLOBAL_K, S))
    mask = pl.pallas_call(
        body,
        grid=(B, S // R),
        in_specs=[
            pl.BlockSpec((1, R, idim), lambda b, rb: (b, rb, 0)),
            pl.BlockSpec((1, S, idim), lambda b, rb: (b, 0, 0)),
        ],
        out_specs=pl.BlockSpec((1, 1, R, S), lambda b, rb: (b, 0, rb, 0)),
        out_shape=jax.ShapeDtypeStruct((B, 1, S, S), jnp.float32),
        compiler_params=pltpu.CompilerParams(
            dimension_semantics=("parallel", "arbitrary")),
    )(qp, kp)
    return mask


# speedup vs baseline: 10.6110x; 1.3907x over previous
import functools

import jax
import jax.numpy as jnp
from jax.experimental import pallas as pl
from jax.experimental.pallas import tpu as pltpu

_LOCAL_WINDOW = 32
_GLOBAL_K = 64
_ROW_BLOCK = 256


def _proj_body(x_ref, wq_ref, wk_ref, qp_ref, kp_ref):
    x = x_ref[0]
    dn = (((1,), (1,)), ((), ()))
    qp_ref[0] = jax.lax.dot_general(x, wq_ref[...], dn,
                                    preferred_element_type=jnp.float32)
    kp_ref[0] = jax.lax.dot_general(x, wk_ref[...], dn,
                                    preferred_element_type=jnp.float32)


def _mask_body(qp_ref, kp_ref, out_ref, *, seq_len, row_block, k):
    rb = pl.program_id(1)
    R, S = row_block, seq_len
    q = qp_ref[0]
    kp = kp_ref[0]
    dn = (((1,), (1,)), ((), ()))
    s = jax.lax.dot_general(q, kp, dn, preferred_element_type=jnp.float32)
    s = jnp.maximum(s, 0.0)
    bits = jax.lax.bitcast_convert_type(s, jnp.int32) & 0x7FFFFFFF

    hi16 = (bits >> 16).astype(jnp.int16)
    lo16 = ((bits & 0xFFFF) - 32768).astype(jnp.int16)

    def hsum(m):
        v = m.astype(jnp.int16)
        w = S
        while w > 128:
            half = w // 2
            v = v[:, :half] + v[:, half:w]
            w = half
        return jnp.sum(v.astype(jnp.int32), axis=1, keepdims=True)

    loA = jnp.zeros((R, 1), jnp.int32)
    hiA = jnp.full((R, 1), jnp.int32(32768))

    def stepA(_, c):
        lo, hi = c
        mid = lo + ((hi - lo) >> 1)
        ok = hsum(hi16 >= mid.astype(jnp.int16)) >= k
        return jnp.where(ok, mid, lo), jnp.where(ok, hi, mid)

    t16, _ = jax.lax.fori_loop(0, 15, stepA, (loA, hiA))

    t16_16 = t16.astype(jnp.int16)
    pref_gt = hi16 > t16_16
    pref_tie = hi16 == t16_16
    g_pref = hsum(pref_gt)
    z16 = jnp.where(pref_tie, lo16, jnp.int16(-32768))

    loB = jnp.full((R, 1), jnp.int32(-32768))
    hiB = jnp.full((R, 1), jnp.int32(32768))

    def stepB(_, c):
        lo, hi = c
        mid = lo + ((hi - lo) >> 1)
        ok = (g_pref + hsum(z16 >= mid.astype(jnp.int16))) >= k
        return jnp.where(ok, mid, lo), jnp.where(ok, hi, mid)

    tl, _ = jax.lax.fori_loop(0, 16, stepB, (loB, hiB))
    tl16 = tl.astype(jnp.int16)

    lo_gt = pref_tie & (lo16 > tl16)
    n_gt = g_pref + hsum(lo_gt)
    needed = k - n_gt
    tie16 = pref_tie & (lo16 == tl16)
    col16 = jax.lax.broadcasted_iota(jnp.int16, (R, S), 1)
    z_idx = jnp.where(tie16, col16, jnp.int16(32767))

    loC = jnp.zeros((R, 1), jnp.int32)
    hiC = jnp.full((R, 1), jnp.int32(S))

    def stepC(_, c):
        lo2, hi2 = c
        mid = lo2 + ((hi2 - lo2) >> 1)
        ok = hsum(z_idx < mid.astype(jnp.int16)) >= needed
        return jnp.where(ok, lo2, mid), jnp.where(ok, mid, hi2)

    _, cut = jax.lax.fori_loop(0, 11, stepC, (loC, hiC))

    row16 = ((rb * R).astype(jnp.int16)
             + jax.lax.broadcasted_iota(jnp.int16, (R, S), 0))
    local16 = (col16 <= row16) & (col16 >= row16 - jnp.int16(_LOCAL_WINDOW - 1))
    final16 = (pref_gt | lo_gt | (tie16 & (col16 < cut.astype(jnp.int16)))
               | local16)
    f = final16.astype(jnp.int16).astype(jnp.float32)
    out_ref[0, 0] = jnp.where(f > 0.5, jnp.float32(0.0), jnp.float32(-jnp.inf))


def kernel(x, Wq, Wk):
    B, S, D = x.shape
    idim = Wq.shape[0]
    qp, kp = pl.pallas_call(
        _proj_body,
        grid=(B,),
        in_specs=[
            pl.BlockSpec((1, S, D), lambda b: (b, 0, 0)),
            pl.BlockSpec((idim, D), lambda b: (0, 0)),
            pl.BlockSpec((idim, D), lambda b: (0, 0)),
        ],
        out_specs=[
            pl.BlockSpec((1, S, idim), lambda b: (b, 0, 0)),
            pl.BlockSpec((1, S, idim), lambda b: (b, 0, 0)),
        ],
        out_shape=[
            jax.ShapeDtypeStruct((B, S, idim), jnp.float32),
            jax.ShapeDtypeStruct((B, S, idim), jnp.float32),
        ],
        compiler_params=pltpu.CompilerParams(
            dimension_semantics=("arbitrary",)),
    )(x, Wq, Wk)

    R = _ROW_BLOCK
    body = functools.partial(_mask_body, seq_len=S, row_block=R,
                             k=min(_GLOBAL_K, S))
    mask = pl.pallas_call(
        body,
        grid=(B, S // R),
        in_specs=[
            pl.BlockSpec((1, R, idim), lambda b, rb: (b, rb, 0)),
            pl.BlockSpec((1, S, idim), lambda b, rb: (b, 0, 0)),
        ],
        out_specs=pl.BlockSpec((1, 1, R, S), lambda b, rb: (b, 0, rb, 0)),
        out_shape=jax.ShapeDtypeStruct((B, 1, S, S), jnp.float32),
        compiler_params=pltpu.CompilerParams(
            dimension_semantics=("parallel", "arbitrary")),
    )(qp, kp)
    return mask


# speedup vs baseline: 11.7607x; 1.1084x over previous
import functools

import jax
import jax.numpy as jnp
from jax.experimental import pallas as pl
from jax.experimental.pallas import tpu as pltpu

_LOCAL_WINDOW = 32
_GLOBAL_K = 64
_ROW_BLOCK = 256


def _proj_body(x_ref, wq_ref, wk_ref, qp_ref, kp_ref):
    x = x_ref[0]
    dn = (((1,), (1,)), ((), ()))
    qp_ref[0] = jax.lax.dot_general(x, wq_ref[...], dn,
                                    preferred_element_type=jnp.float32)
    kp_ref[0] = jax.lax.dot_general(x, wk_ref[...], dn,
                                    preferred_element_type=jnp.float32)


def _mask_body(qp_ref, kp_ref, out_ref, *, seq_len, row_block, k):
    rb = pl.program_id(1)
    R, S = row_block, seq_len
    q = qp_ref[0]
    kp = kp_ref[0]
    dn = (((1,), (1,)), ((), ()))
    s = jax.lax.dot_general(q, kp, dn, preferred_element_type=jnp.float32)
    s = jnp.maximum(s, 0.0)
    bits = jax.lax.bitcast_convert_type(s, jnp.int32) & 0x7FFFFFFF

    hi16 = (bits >> 16).astype(jnp.int16)
    lo16 = bits.astype(jnp.int16) ^ jnp.int16(-32768)

    def hsum(m):
        v = m.astype(jnp.int16)
        w = S
        while w > 128:
            half = w // 2
            v = v[:, :half] + v[:, half:w]
            w = half
        return jnp.sum(v.astype(jnp.int32), axis=1, keepdims=True)

    def stepA(i, lo):
        mid = lo + (jnp.int32(16384) >> i)
        ok = hsum(hi16 >= mid.astype(jnp.int16)) >= k
        return jnp.where(ok, mid, lo)

    t16 = jax.lax.fori_loop(0, 15, stepA, jnp.zeros((R, 1), jnp.int32))

    t16_16 = t16.astype(jnp.int16)
    pref_gt = hi16 > t16_16
    pref_tie = hi16 == t16_16
    g_pref = hsum(pref_gt)
    z16 = jnp.where(pref_tie, lo16, jnp.int16(-32768))

    def stepB(i, lo):
        mid = lo + (jnp.int32(32768) >> i)
        ok = (g_pref + hsum(z16 >= mid.astype(jnp.int16))) >= k
        return jnp.where(ok, mid, lo)

    tl = jax.lax.fori_loop(0, 16, stepB,
                           jnp.full((R, 1), jnp.int32(-32768)))
    tl16 = tl.astype(jnp.int16)

    lo_gt = pref_tie & (lo16 > tl16)
    n_gt = g_pref + hsum(lo_gt)
    needed = k - n_gt
    tie16 = pref_tie & (lo16 == tl16)
    col16 = jax.lax.broadcasted_iota(jnp.int16, (R, S), 1)
    z_idx = jnp.where(tie16, col16, jnp.int16(32767))

    n_bits_s = max(1, (S - 1).bit_length())

    def stepC(i, x):
        mid = x + (jnp.int32(S >> 1) >> i)
        ok = hsum(z_idx < mid.astype(jnp.int16)) < needed
        return jnp.where(ok, mid, x)

    x_cut = jax.lax.fori_loop(0, n_bits_s, stepC,
                              jnp.zeros((R, 1), jnp.int32))
    cut = x_cut + 1

    row16 = ((rb * R).astype(jnp.int16)
             + jax.lax.broadcasted_iota(jnp.int16, (R, S), 0))
    local16 = (col16 <= row16) & (col16 >= row16 - jnp.int16(_LOCAL_WINDOW - 1))
    final16 = (pref_gt | lo_gt | (tie16 & (col16 < cut.astype(jnp.int16)))
               | local16)
    h16 = jnp.where(final16, jnp.int16(0), jnp.int16(-128))
    out_ref[0, 0] = jax.lax.bitcast_convert_type(
        h16.astype(jnp.int32) << 16, jnp.float32)


def kernel(x, Wq, Wk):
    B, S, D = x.shape
    idim = Wq.shape[0]
    qp, kp = pl.pallas_call(
        _proj_body,
        grid=(B,),
        in_specs=[
            pl.BlockSpec((1, S, D), lambda b: (b, 0, 0)),
            pl.BlockSpec((idim, D), lambda b: (0, 0)),
            pl.BlockSpec((idim, D), lambda b: (0, 0)),
        ],
        out_specs=[
            pl.BlockSpec((1, S, idim), lambda b: (b, 0, 0)),
            pl.BlockSpec((1, S, idim), lambda b: (b, 0, 0)),
        ],
        out_shape=[
            jax.ShapeDtypeStruct((B, S, idim), jnp.float32),
            jax.ShapeDtypeStruct((B, S, idim), jnp.float32),
        ],
        compiler_params=pltpu.CompilerParams(
            dimension_semantics=("arbitrary",)),
    )(x, Wq, Wk)

    R = _ROW_BLOCK
    body = functools.partial(_mask_body, seq_len=S, row_block=R,
                             k=min(_GLOBAL_K, S))
    mask = pl.pallas_call(
        body,
        grid=(B, S // R),
        in_specs=[
            pl.BlockSpec((1, R, idim), lambda b, rb: (b, rb, 0)),
            pl.BlockSpec((1, S, idim), lambda b, rb: (b, 0, 0)),
        ],
        out_specs=pl.BlockSpec((1, 1, R, S), lambda b, rb: (b, 0, rb, 0)),
        out_shape=jax.ShapeDtypeStruct((B, 1, S, S), jnp.float32),
        compiler_params=pltpu.CompilerParams(
            dimension_semantics=("parallel", "arbitrary")),
    )(qp, kp)
    return mask


# speedup vs baseline: 13.3647x; 1.1364x over previous
import functools

import jax
import jax.numpy as jnp
from jax.experimental import pallas as pl
from jax.experimental.pallas import tpu as pltpu

_LOCAL_WINDOW = 32
_GLOBAL_K = 64
_ROW_BLOCK = 512


def _proj_body(x_ref, wq_ref, wk_ref, qp_ref, kp_ref):
    x = x_ref[0]
    dn = (((1,), (1,)), ((), ()))
    qp_ref[0] = jax.lax.dot_general(x, wq_ref[...], dn,
                                    preferred_element_type=jnp.float32)
    kp_ref[0] = jax.lax.dot_general(x, wk_ref[...], dn,
                                    preferred_element_type=jnp.float32)


def _mask_body(qp_ref, kp_ref, out_ref, *, seq_len, row_block, k):
    rb = pl.program_id(1)
    R, S = row_block, seq_len
    q = qp_ref[0]
    kp = kp_ref[0]
    dn = (((1,), (1,)), ((), ()))
    s = jax.lax.dot_general(q, kp, dn, preferred_element_type=jnp.float32)
    s = jnp.maximum(s, 0.0)
    bits = jax.lax.bitcast_convert_type(s, jnp.int32) & 0x7FFFFFFF

    hi16 = (bits >> 16).astype(jnp.int16)
    lo16 = bits.astype(jnp.int16) ^ jnp.int16(-32768)

    def hsum(m):
        v = m.astype(jnp.int16)
        w = S
        while w > 128:
            half = w // 2
            v = v[:, :half] + v[:, half:w]
            w = half
        return jnp.sum(v.astype(jnp.int32), axis=1, keepdims=True)

    def stepA(i, lo):
        mid = lo + (jnp.int32(16384) >> i)
        ok = hsum(hi16 >= mid.astype(jnp.int16)) >= k
        return jnp.where(ok, mid, lo)

    t16 = jax.lax.fori_loop(0, 15, stepA, jnp.zeros((R, 1), jnp.int32))

    t16_16 = t16.astype(jnp.int16)
    pref_gt = hi16 > t16_16
    pref_tie = hi16 == t16_16
    g_pref = hsum(pref_gt)
    z16 = jnp.where(pref_tie, lo16, jnp.int16(-32768))

    def stepB(i, lo):
        mid = lo + (jnp.int32(32768) >> i)
        ok = (g_pref + hsum(z16 >= mid.astype(jnp.int16))) >= k
        return jnp.where(ok, mid, lo)

    tl = jax.lax.fori_loop(0, 16, stepB,
                           jnp.full((R, 1), jnp.int32(-32768)))
    tl16 = tl.astype(jnp.int16)

    lo_gt = pref_tie & (lo16 > tl16)
    n_gt = g_pref + hsum(lo_gt)
    needed = k - n_gt
    tie16 = pref_tie & (lo16 == tl16)
    col16 = jax.lax.broadcasted_iota(jnp.int16, (R, S), 1)
    z_idx = jnp.where(tie16, col16, jnp.int16(32767))

    n_bits_s = max(1, (S - 1).bit_length())

    def stepC(i, x):
        mid = x + (jnp.int32(S >> 1) >> i)
        ok = hsum(z_idx < mid.astype(jnp.int16)) < needed
        return jnp.where(ok, mid, x)

    x_cut = jax.lax.fori_loop(0, n_bits_s, stepC,
                              jnp.zeros((R, 1), jnp.int32))
    cut = x_cut + 1

    row16 = ((rb * R).astype(jnp.int16)
             + jax.lax.broadcasted_iota(jnp.int16, (R, S), 0))
    local16 = (col16 <= row16) & (col16 >= row16 - jnp.int16(_LOCAL_WINDOW - 1))
    final16 = (pref_gt | lo_gt | (tie16 & (col16 < cut.astype(jnp.int16)))
               | local16)
    h16 = jnp.where(final16, jnp.int16(0), jnp.int16(-128))
    out_ref[0, 0] = jax.lax.bitcast_convert_type(
        h16.astype(jnp.int32) << 16, jnp.float32)


def kernel(x, Wq, Wk):
    B, S, D = x.shape
    idim = Wq.shape[0]
    qp, kp = pl.pallas_call(
        _proj_body,
        grid=(B,),
        in_specs=[
            pl.BlockSpec((1, S, D), lambda b: (b, 0, 0)),
            pl.BlockSpec((idim, D), lambda b: (0, 0)),
            pl.BlockSpec((idim, D), lambda b: (0, 0)),
        ],
        out_specs=[
            pl.BlockSpec((1, S, idim), lambda b: (b, 0, 0)),
            pl.BlockSpec((1, S, idim), lambda b: (b, 0, 0)),
        ],
        out_shape=[
            jax.ShapeDtypeStruct((B, S, idim), jnp.float32),
            jax.ShapeDtypeStruct((B, S, idim), jnp.float32),
        ],
        compiler_params=pltpu.CompilerParams(
            dimension_semantics=("arbitrary",)),
    )(x, Wq, Wk)

    R = _ROW_BLOCK
    body = functools.partial(_mask_body, seq_len=S, row_block=R,
                             k=min(_GLOBAL_K, S))
    mask = pl.pallas_call(
        body,
        grid=(B, S // R),
        in_specs=[
            pl.BlockSpec((1, R, idim), lambda b, rb: (b, rb, 0)),
            pl.BlockSpec((1, S, idim), lambda b, rb: (b, 0, 0)),
        ],
        out_specs=pl.BlockSpec((1, 1, R, S), lambda b, rb: (b, 0, rb, 0)),
        out_shape=jax.ShapeDtypeStruct((B, 1, S, S), jnp.float32),
        compiler_params=pltpu.CompilerParams(
            dimension_semantics=("parallel", "arbitrary")),
    )(qp, kp)
    return mask


# speedup vs baseline: 14.1777x; 1.0608x over previous
import functools

import jax
import jax.numpy as jnp
from jax.experimental import pallas as pl
from jax.experimental.pallas import tpu as pltpu

_LOCAL_WINDOW = 32
_GLOBAL_K = 64
_ROW_BLOCK = 1024


def _proj_body(x_ref, wq_ref, wk_ref, qp_ref, kp_ref):
    x = x_ref[0]
    dn = (((1,), (1,)), ((), ()))
    qp_ref[0] = jax.lax.dot_general(x, wq_ref[...], dn,
                                    preferred_element_type=jnp.float32)
    kp_ref[0] = jax.lax.dot_general(x, wk_ref[...], dn,
                                    preferred_element_type=jnp.float32)


def _mask_body(qp_ref, kp_ref, out_ref, *, seq_len, row_block, k):
    rb = pl.program_id(1)
    R, S = row_block, seq_len
    q = qp_ref[0]
    kp = kp_ref[0]
    dn = (((1,), (1,)), ((), ()))
    s = jax.lax.dot_general(q, kp, dn, preferred_element_type=jnp.float32)
    s = jnp.maximum(s, 0.0)
    bits = jax.lax.bitcast_convert_type(s, jnp.int32) & 0x7FFFFFFF

    hi16 = (bits >> 16).astype(jnp.int16)
    lo16 = bits.astype(jnp.int16) ^ jnp.int16(-32768)

    def hsum(m):
        v = m.astype(jnp.int16)
        w = S
        while w > 128:
            half = w // 2
            v = v[:, :half] + v[:, half:w]
            w = half
        return jnp.sum(v.astype(jnp.int32), axis=1, keepdims=True)

    def stepA(i, lo):
        mid = lo + (jnp.int32(16384) >> i)
        ok = hsum(hi16 >= mid.astype(jnp.int16)) >= k
        return jnp.where(ok, mid, lo)

    t16 = jax.lax.fori_loop(0, 15, stepA, jnp.zeros((R, 1), jnp.int32))

    t16_16 = t16.astype(jnp.int16)
    pref_gt = hi16 > t16_16
    pref_tie = hi16 == t16_16
    g_pref = hsum(pref_gt)
    z16 = jnp.where(pref_tie, lo16, jnp.int16(-32768))

    def stepB(i, lo):
        mid = lo + (jnp.int32(32768) >> i)
        ok = (g_pref + hsum(z16 >= mid.astype(jnp.int16))) >= k
        return jnp.where(ok, mid, lo)

    tl = jax.lax.fori_loop(0, 16, stepB,
                           jnp.full((R, 1), jnp.int32(-32768)))
    tl16 = tl.astype(jnp.int16)

    lo_gt = pref_tie & (lo16 > tl16)
    n_gt = g_pref + hsum(lo_gt)
    needed = k - n_gt
    tie16 = pref_tie & (lo16 == tl16)
    col16 = jax.lax.broadcasted_iota(jnp.int16, (R, S), 1)
    z_idx = jnp.where(tie16, col16, jnp.int16(32767))

    n_bits_s = max(1, (S - 1).bit_length())

    def stepC(i, x):
        mid = x + (jnp.int32(S >> 1) >> i)
        ok = hsum(z_idx < mid.astype(jnp.int16)) < needed
        return jnp.where(ok, mid, x)

    x_cut = jax.lax.fori_loop(0, n_bits_s, stepC,
                              jnp.zeros((R, 1), jnp.int32))
    cut = x_cut + 1

    row16 = ((rb * R).astype(jnp.int16)
             + jax.lax.broadcasted_iota(jnp.int16, (R, S), 0))
    local16 = (col16 <= row16) & (col16 >= row16 - jnp.int16(_LOCAL_WINDOW - 1))
    final16 = (pref_gt | lo_gt | (tie16 & (col16 < cut.astype(jnp.int16)))
               | local16)
    h16 = jnp.where(final16, jnp.int16(0), jnp.int16(-128))
    out_ref[0, 0] = jax.lax.bitcast_convert_type(
        h16.astype(jnp.int32) << 16, jnp.float32)


def kernel(x, Wq, Wk):
    B, S, D = x.shape
    idim = Wq.shape[0]
    qp, kp = pl.pallas_call(
        _proj_body,
        grid=(B,),
        in_specs=[
            pl.BlockSpec((1, S, D), lambda b: (b, 0, 0)),
            pl.BlockSpec((idim, D), lambda b: (0, 0)),
            pl.BlockSpec((idim, D), lambda b: (0, 0)),
        ],
        out_specs=[
            pl.BlockSpec((1, S, idim), lambda b: (b, 0, 0)),
            pl.BlockSpec((1, S, idim), lambda b: (b, 0, 0)),
        ],
        out_shape=[
            jax.ShapeDtypeStruct((B, S, idim), jnp.float32),
            jax.ShapeDtypeStruct((B, S, idim), jnp.float32),
        ],
        compiler_params=pltpu.CompilerParams(
            dimension_semantics=("arbitrary",)),
    )(x, Wq, Wk)

    R = _ROW_BLOCK
    body = functools.partial(_mask_body, seq_len=S, row_block=R,
                             k=min(_GLOBAL_K, S))
    mask = pl.pallas_call(
        body,
        grid=(B, S // R),
        in_specs=[
            pl.BlockSpec((1, R, idim), lambda b, rb: (b, rb, 0)),
            pl.BlockSpec((1, S, idim), lambda b, rb: (b, 0, 0)),
        ],
        out_specs=pl.BlockSpec((1, 1, R, S), lambda b, rb: (b, 0, rb, 0)),
        out_shape=jax.ShapeDtypeStruct((B, 1, S, S), jnp.float32),
        compiler_params=pltpu.CompilerParams(
            dimension_semantics=("parallel", "arbitrary")),
    )(qp, kp)
    return mask


# speedup vs baseline: 15.4221x; 1.0878x over previous
import functools

import jax
import jax.numpy as jnp
from jax.experimental import pallas as pl
from jax.experimental.pallas import tpu as pltpu

_LOCAL_WINDOW = 32
_GLOBAL_K = 64
_ROW_BLOCK = 1024


def _proj_body(x_ref, wq_ref, wk_ref, qp_ref, kp_ref):
    x = x_ref[0]
    dn = (((1,), (1,)), ((), ()))
    qp_ref[0] = jax.lax.dot_general(x, wq_ref[...], dn,
                                    preferred_element_type=jnp.float32)
    kp_ref[0] = jax.lax.dot_general(x, wk_ref[...], dn,
                                    preferred_element_type=jnp.float32)


def _mask_body(qp_ref, kp_ref, out_ref, *, seq_len, row_block, k):
    rb = pl.program_id(1)
    R, S = row_block, seq_len
    q = qp_ref[0]
    kp = kp_ref[0]
    dn = (((1,), (1,)), ((), ()))
    s = jax.lax.dot_general(q, kp, dn, preferred_element_type=jnp.float32)
    s = jnp.maximum(s, 0.0)
    bits = jax.lax.bitcast_convert_type(s, jnp.int32) & 0x7FFFFFFF

    hi16 = (bits >> 16).astype(jnp.int16)
    lo16 = bits.astype(jnp.int16) ^ jnp.int16(-32768)

    def hsum(m):
        v = m.astype(jnp.int16)
        w = S
        while w > 128:
            half = w // 2
            v = v[:, :half] + v[:, half:w]
            w = half
        s32 = jnp.sum(v.astype(jnp.int32), axis=1, keepdims=True)
        return s32.astype(jnp.int16)

    k16 = jnp.int16(k)

    def stepA(i, lo):
        mid = lo + (jnp.int32(16384) >> i).astype(jnp.int16)
        ok = hsum(hi16 >= mid) >= k16
        return jnp.where(ok, mid, lo)

    t16_16 = jax.lax.fori_loop(0, 15, stepA, jnp.zeros((R, 1), jnp.int16))

    pref_gt = hi16 > t16_16
    pref_tie = hi16 == t16_16
    g_pref = hsum(pref_gt)
    k_eff = k16 - g_pref
    z16 = jnp.where(pref_tie, lo16, jnp.int16(-32768))

    ok0 = hsum(z16 >= jnp.int16(0)) >= k_eff
    lo0 = jnp.where(ok0, jnp.int16(0), jnp.int16(-32768))

    def stepB(i, lo):
        mid = lo + (jnp.int32(16384) >> i).astype(jnp.int16)
        ok = hsum(z16 >= mid) >= k_eff
        return jnp.where(ok, mid, lo)

    tl16 = jax.lax.fori_loop(0, 15, stepB, lo0)

    lo_gt = pref_tie & (lo16 > tl16)
    needed = k_eff - hsum(lo_gt)
    tie16 = pref_tie & (lo16 == tl16)
    col16 = jax.lax.broadcasted_iota(jnp.int16, (R, S), 1)
    z_idx = jnp.where(tie16, col16, jnp.int16(32767))

    n_bits_s = max(1, (S - 1).bit_length())

    def stepC(i, x):
        mid = x + (jnp.int32(S >> 1) >> i).astype(jnp.int16)
        ok = hsum(z_idx < mid) < needed
        return jnp.where(ok, mid, x)

    x_cut = jax.lax.fori_loop(0, n_bits_s, stepC,
                              jnp.zeros((R, 1), jnp.int16))
    cut = x_cut + jnp.int16(1)

    row16 = ((rb * R).astype(jnp.int16)
             + jax.lax.broadcasted_iota(jnp.int16, (R, S), 0))
    local16 = (col16 <= row16) & (col16 >= row16 - jnp.int16(_LOCAL_WINDOW - 1))
    final16 = (pref_gt | lo_gt | (tie16 & (col16 < cut.astype(jnp.int16)))
               | local16)
    h16 = jnp.where(final16, jnp.int16(0), jnp.int16(-128))
    out_ref[0, 0] = jax.lax.bitcast_convert_type(
        h16.astype(jnp.int32) << 16, jnp.float32)


def kernel(x, Wq, Wk):
    B, S, D = x.shape
    idim = Wq.shape[0]
    qp, kp = pl.pallas_call(
        _proj_body,
        grid=(B,),
        in_specs=[
            pl.BlockSpec((1, S, D), lambda b: (b, 0, 0)),
            pl.BlockSpec((idim, D), lambda b: (0, 0)),
            pl.BlockSpec((idim, D), lambda b: (0, 0)),
        ],
        out_specs=[
            pl.BlockSpec((1, S, idim), lambda b: (b, 0, 0)),
            pl.BlockSpec((1, S, idim), lambda b: (b, 0, 0)),
        ],
        out_shape=[
            jax.ShapeDtypeStruct((B, S, idim), jnp.float32),
            jax.ShapeDtypeStruct((B, S, idim), jnp.float32),
        ],
        compiler_params=pltpu.CompilerParams(
            dimension_semantics=("arbitrary",)),
    )(x, Wq, Wk)

    R = _ROW_BLOCK
    body = functools.partial(_mask_body, seq_len=S, row_block=R,
                             k=min(_GLOBAL_K, S))
    mask = pl.pallas_call(
        body,
        grid=(B, S // R),
        in_specs=[
            pl.BlockSpec((1, R, idim), lambda b, rb: (b, rb, 0)),
            pl.BlockSpec((1, S, idim), lambda b, rb: (b, 0, 0)),
        ],
        out_specs=pl.BlockSpec((1, 1, R, S), lambda b, rb: (b, 0, rb, 0)),
        out_shape=jax.ShapeDtypeStruct((B, 1, S, S), jnp.float32),
        compiler_params=pltpu.CompilerParams(
            dimension_semantics=("parallel", "arbitrary")),
    )(qp, kp)
    return mask


# speedup vs baseline: 15.6437x; 1.0144x over previous
import functools

import jax
import jax.numpy as jnp
from jax.experimental import pallas as pl
from jax.experimental.pallas import tpu as pltpu

_LOCAL_WINDOW = 32
_GLOBAL_K = 64
_ROW_BLOCK = 1024


def _mask_body(x_ref, wq_ref, wk_ref, out_ref, kp_ref, *, seq_len, row_block,
               k):
    rb = pl.program_id(1)
    R, S = row_block, seq_len
    dn = (((1,), (1,)), ((), ()))

    @pl.when(rb == 0)
    def _():
        kp_ref[...] = jax.lax.dot_general(
            x_ref[0], wk_ref[...], dn, preferred_element_type=jnp.float32)

    x_rows = x_ref[0, pl.ds(rb * R, R), :]
    q = jax.lax.dot_general(x_rows, wq_ref[...], dn,
                            preferred_element_type=jnp.float32)
    kp = kp_ref[...]
    s = jax.lax.dot_general(q, kp, dn, preferred_element_type=jnp.float32)
    s = jnp.maximum(s, 0.0)
    bits = jax.lax.bitcast_convert_type(s, jnp.int32) & 0x7FFFFFFF

    hi16 = (bits >> 16).astype(jnp.int16)
    lo16 = bits.astype(jnp.int16) ^ jnp.int16(-32768)

    def hsum(m):
        v = m.astype(jnp.int16)
        w = S
        while w > 128:
            half = w // 2
            v = v[:, :half] + v[:, half:w]
            w = half
        s32 = jnp.sum(v.astype(jnp.int32), axis=1, keepdims=True)
        return s32.astype(jnp.int16)

    k16 = jnp.int16(k)

    def stepA(i, lo):
        mid = lo + (jnp.int32(16384) >> i).astype(jnp.int16)
        ok = hsum(hi16 >= mid) >= k16
        return jnp.where(ok, mid, lo)

    t16_16 = jax.lax.fori_loop(0, 15, stepA, jnp.zeros((R, 1), jnp.int16))

    pref_gt = hi16 > t16_16
    pref_tie = hi16 == t16_16
    g_pref = hsum(pref_gt)
    k_eff = k16 - g_pref
    z16 = jnp.where(pref_tie, lo16, jnp.int16(-32768))

    ok0 = hsum(z16 >= jnp.int16(0)) >= k_eff
    lo0 = jnp.where(ok0, jnp.int16(0), jnp.int16(-32768))

    def stepB(i, lo):
        mid = lo + (jnp.int32(16384) >> i).astype(jnp.int16)
        ok = hsum(z16 >= mid) >= k_eff
        return jnp.where(ok, mid, lo)

    tl16 = jax.lax.fori_loop(0, 15, stepB, lo0)

    lo_gt = pref_tie & (lo16 > tl16)
    needed = k_eff - hsum(lo_gt)
    tie16 = pref_tie & (lo16 == tl16)
    col16 = jax.lax.broadcasted_iota(jnp.int16, (R, S), 1)
    z_idx = jnp.where(tie16, col16, jnp.int16(32767))

    n_bits_s = max(1, (S - 1).bit_length())

    def stepC(i, x):
        mid = x + (jnp.int32(S >> 1) >> i).astype(jnp.int16)
        ok = hsum(z_idx < mid) < needed
        return jnp.where(ok, mid, x)

    x_cut = jax.lax.fori_loop(0, n_bits_s, stepC,
                              jnp.zeros((R, 1), jnp.int16))
    cut = x_cut + jnp.int16(1)

    row16 = ((rb * R).astype(jnp.int16)
             + jax.lax.broadcasted_iota(jnp.int16, (R, S), 0))
    local16 = (col16 <= row16) & (col16 >= row16 - jnp.int16(_LOCAL_WINDOW - 1))
    final16 = (pref_gt | lo_gt | (tie16 & (col16 < cut.astype(jnp.int16)))
               | local16)
    h16 = jnp.where(final16, jnp.int16(0), jnp.int16(-128))
    out_ref[0, 0] = jax.lax.bitcast_convert_type(
        h16.astype(jnp.int32) << 16, jnp.float32)


def kernel(x, Wq, Wk):
    B, S, D = x.shape
    idim = Wq.shape[0]
    R = _ROW_BLOCK
    body = functools.partial(_mask_body, seq_len=S, row_block=R,
                             k=min(_GLOBAL_K, S))
    mask = pl.pallas_call(
        body,
        grid=(B, S // R),
        in_specs=[
            pl.BlockSpec((1, S, D), lambda b, rb: (b, 0, 0)),
            pl.BlockSpec((idim, D), lambda b, rb: (0, 0)),
            pl.BlockSpec((idim, D), lambda b, rb: (0, 0)),
        ],
        out_specs=pl.BlockSpec((1, 1, R, S), lambda b, rb: (b, 0, rb, 0)),
        out_shape=jax.ShapeDtypeStruct((B, 1, S, S), jnp.float32),
        scratch_shapes=[pltpu.VMEM((S, idim), jnp.float32)],
        compiler_params=pltpu.CompilerParams(
            dimension_semantics=("arbitrary", "arbitrary")),
    )(x, Wq, Wk)
    return mask
